# Initial kernel scaffold; baseline (speedup 1.0000x reference)
import jax
import jax.numpy as jnp
from jax.experimental import pallas as pl


def kernel(x, edge_index, e, W1, b1, We, be, W2, b2, W3, b3, W4, b4):
    raise NotImplementedError("write your pallas kernel here")


# speedup vs baseline: 2.5157x; 2.5157x over previous
import jax
import jax.numpy as jnp
from jax import lax
from jax.experimental import pallas as pl
from jax.experimental.pallas import tpu as pltpu
from jax.experimental.pallas import tpu_sc as plsc

N = 10000
E = 320000
F = 128
H = 256
R = 1000
NSUB = 16
NCORE = 2
K = 80
ZB = 200
NBLK = N // ZB
BLK_ITERS = -(-NBLK // NSUB)

ES1 = E // NSUB
NCHUNK1 = ES1 // K
ES2 = E // (NCORE * NSUB)
NCHUNK2 = ES2 // K


def _prep_body(x_ref, we_ref, be_ref, a_ref, b_ref):
    xb = x_ref[...]
    wa = we_ref[0:F, :]
    wb = we_ref[F : 2 * F, :]
    a_ref[0] = (
        jnp.dot(xb, wa - wb, preferred_element_type=jnp.float32) + be_ref[...]
    )
    b_ref[0] = jnp.dot(xb, wb, preferred_element_type=jnp.float32)


_prep_call = pl.pallas_call(
    _prep_body,
    grid=(N // R, 2),
    in_specs=[
        pl.BlockSpec((R, F), lambda i, h: (i, 0)),
        pl.BlockSpec((2 * F, F), lambda i, h: (0, h)),
        pl.BlockSpec((1, F), lambda i, h: (0, h)),
    ],
    out_specs=[
        pl.BlockSpec((1, R, F), lambda i, h: (h, i, 0)),
        pl.BlockSpec((1, R, F), lambda i, h: (h, i, 0)),
    ],
    out_shape=[
        jax.ShapeDtypeStruct((2, N, F), jnp.float32),
        jax.ShapeDtypeStruct((2, N, F), jnp.float32),
    ],
)


def _sc_eagg_body(
    src_hbm, dst_hbm, a_hbm, b_hbm, eagg_hbm,
    idx_src, idx_dst, idx_sa, idx_da, buf_a, buf_b, zbuf, eagg_sh, sem,
):
    c = lax.axis_index("c")
    s = lax.axis_index("s")
    cn = c * N

    z16 = jnp.zeros((16,), jnp.float32)

    def zrow(i, carry):
        for j in range(F // 16):
            zbuf[i, pl.ds(j * 16, 16)] = z16
        return carry

    lax.fori_loop(0, ZB, zrow, 0)

    def zblk(t, carry):
        b = s + NSUB * t

        @pl.when(b < NBLK)
        def _():
            pltpu.sync_copy(zbuf, eagg_sh.at[pl.ds(b * ZB, ZB)])

        return carry

    lax.fori_loop(0, BLK_ITERS, zblk, 0)
    plsc.subcore_barrier()

    def chunk(g, carry):
        off = s * ES1 + g * K
        pltpu.sync_copy(src_hbm.at[pl.ds(off, K)], idx_src)
        pltpu.sync_copy(dst_hbm.at[pl.ds(off, K)], idx_dst)
        for j in range(K // 16):
            sl = pl.ds(j * 16, 16)
            idx_sa[sl] = idx_src[sl] + cn
            idx_da[sl] = idx_dst[sl] + cn
        cp_a = pltpu.async_copy(a_hbm.at[idx_da], buf_a, sem)
        cp_b = pltpu.async_copy(b_hbm.at[idx_sa], buf_b, sem)
        cp_a.wait()
        cp_b.wait()

        def row(k, carry2):
            for j in range(F // 16):
                sl = pl.ds(j * 16, 16)
                buf_a[k, sl] = jnp.maximum(buf_a[k, sl] + buf_b[k, sl], 0.0)
            return carry2

        lax.fori_loop(0, K, row, 0)
        pltpu.sync_copy(buf_a, eagg_sh.at[idx_dst], add=True)
        return carry

    lax.fori_loop(0, NCHUNK1, chunk, 0)
    plsc.subcore_barrier()

    def wblk(t, carry):
        b = s + NSUB * t

        @pl.when(b < NBLK)
        def _():
            r0 = b * ZB
            pltpu.sync_copy(eagg_sh.at[pl.ds(r0, ZB)], eagg_hbm.at[pl.ds(cn + r0, ZB)])

        return carry

    lax.fori_loop(0, BLK_ITERS, wblk, 0)


_sc_eagg_call = pl.kernel(
    _sc_eagg_body,
    out_type=jax.ShapeDtypeStruct((2 * N, F), jnp.float32),
    mesh=plsc.VectorSubcoreMesh(core_axis_name="c", subcore_axis_name="s"),
    scratch_types=[
        pltpu.VMEM((K,), jnp.int32),
        pltpu.VMEM((K,), jnp.int32),
        pltpu.VMEM((K,), jnp.int32),
        pltpu.VMEM((K,), jnp.int32),
        pltpu.VMEM((K, F), jnp.float32),
        pltpu.VMEM((K, F), jnp.float32),
        pltpu.VMEM((ZB, F), jnp.float32),
        pltpu.VMEM_SHARED((N, F), jnp.float32),
        pltpu.SemaphoreType.DMA,
    ],
)


def _sc_agg_body(
    src_hbm, dst_hbm, x_hbm, agg_hbm,
    idx_src, idx_dst, buf_x, zbuf, agg_sh, sem,
):
    c = lax.axis_index("c")
    s = lax.axis_index("s")
    cn = c * N

    z16 = jnp.zeros((16,), jnp.float32)

    def zrow(i, carry):
        for j in range(F // 16):
            zbuf[i, pl.ds(j * 16, 16)] = z16
        return carry

    lax.fori_loop(0, ZB, zrow, 0)

    def zblk(t, carry):
        b = s + NSUB * t

        @pl.when(b < NBLK)
        def _():
            pltpu.sync_copy(zbuf, agg_sh.at[pl.ds(b * ZB, ZB)])

        return carry

    lax.fori_loop(0, BLK_ITERS, zblk, 0)
    plsc.subcore_barrier()

    def chunk(g, carry):
        off = c * (E // NCORE) + s * ES2 + g * K
        pltpu.sync_copy(src_hbm.at[pl.ds(off, K)], idx_src)
        pltpu.sync_copy(dst_hbm.at[pl.ds(off, K)], idx_dst)
        pltpu.async_copy(x_hbm.at[idx_src], buf_x, sem).wait()
        pltpu.sync_copy(buf_x, agg_sh.at[idx_dst], add=True)
        return carry

    lax.fori_loop(0, NCHUNK2, chunk, 0)
    plsc.subcore_barrier()

    def wblk(t, carry):
        b = s + NSUB * t

        @pl.when(b < NBLK)
        def _():
            r0 = b * ZB
            pltpu.sync_copy(agg_sh.at[pl.ds(r0, ZB)], agg_hbm.at[pl.ds(cn + r0, ZB)])

        return carry

    lax.fori_loop(0, BLK_ITERS, wblk, 0)


_sc_agg_call = pl.kernel(
    _sc_agg_body,
    out_type=jax.ShapeDtypeStruct((2 * N, F), jnp.float32),
    mesh=plsc.VectorSubcoreMesh(core_axis_name="c", subcore_axis_name="s"),
    scratch_types=[
        pltpu.VMEM((K,), jnp.int32),
        pltpu.VMEM((K,), jnp.int32),
        pltpu.VMEM((K, F), jnp.float32),
        pltpu.VMEM((ZB, F), jnp.float32),
        pltpu.VMEM_SHARED((N, F), jnp.float32),
        pltpu.SemaphoreType.DMA,
    ],
)


def _head_body(
    a0_ref, a1_ref, e0_ref, e1_ref,
    w1_ref, b1_ref, w2_ref, b2_ref, w3_ref, b3_ref, w4_ref, b4_ref,
    out_ref,
):
    f32 = jnp.float32
    agg = a0_ref[...] + a1_ref[...]
    nodes = jnp.maximum(
        jnp.dot(agg, w1_ref[...], preferred_element_type=f32) + b1_ref[...],
        0.0,
    )
    edges = jnp.maximum(
        jnp.dot(e0_ref[...], w2_ref[0:F, :], preferred_element_type=f32)
        + jnp.dot(e1_ref[...], w2_ref[F : 2 * F, :], preferred_element_type=f32)
        + b2_ref[...],
        0.0,
    )
    h3 = jnp.maximum(
        jnp.dot(nodes, w3_ref[0:H, :], preferred_element_type=f32)
        + jnp.dot(edges, w3_ref[H : H + F, :], preferred_element_type=f32)
        + b3_ref[...],
        0.0,
    )
    logits = jnp.dot(h3, w4_ref[...], preferred_element_type=f32) + b4_ref[...]
    out_ref[...] = 1.0 / (1.0 + jnp.exp(-logits))


_head_call = pl.pallas_call(
    _head_body,
    grid=(N // R,),
    in_specs=[
        pl.BlockSpec((R, F), lambda i: (i, 0)),
        pl.BlockSpec((R, F), lambda i: (i + N // R, 0)),
        pl.BlockSpec((R, F), lambda i: (i, 0)),
        pl.BlockSpec((R, F), lambda i: (i + N // R, 0)),
        pl.BlockSpec((F, H), lambda i: (0, 0)),
        pl.BlockSpec((1, H), lambda i: (0, 0)),
        pl.BlockSpec((2 * F, F), lambda i: (0, 0)),
        pl.BlockSpec((1, F), lambda i: (0, 0)),
        pl.BlockSpec((H + F, 32), lambda i: (0, 0)),
        pl.BlockSpec((1, 32), lambda i: (0, 0)),
        pl.BlockSpec((32, 1), lambda i: (0, 0)),
        pl.BlockSpec((1, 1), lambda i: (0, 0)),
    ],
    out_specs=pl.BlockSpec((R, 1), lambda i: (i, 0)),
    out_shape=jax.ShapeDtypeStruct((N, 1), jnp.float32),
)


@jax.jit
def kernel(x, edge_index, e, W1, b1, We, be, W2, b2, W3, b3, W4, b4):
    del e
    src = edge_index[0]
    dst = edge_index[1]
    a2, b2t = _prep_call(x, We, be.reshape(1, H))
    eagg = _sc_eagg_call(src, dst, a2.reshape(2 * N, F), b2t.reshape(2 * N, F))
    aggp = _sc_agg_call(src, dst, x)
    out = _head_call(
        aggp, aggp, eagg, eagg,
        W1, b1.reshape(1, H),
        W2, b2.reshape(1, F),
        W3, b3.reshape(1, 32),
        W4, b4.reshape(1, 1),
    )
    return out


# speedup vs baseline: 5.5904x; 2.2222x over previous
import jax
import jax.numpy as jnp
from jax import lax
from jax.experimental import pallas as pl
from jax.experimental.pallas import tpu as pltpu
from jax.experimental.pallas import tpu_sc as plsc

N = 10000
E = 320000
F = 128
H = 256
R = 1000
NSUB = 16
NCORE = 2
K = 80
ZB = 80
NBLK = N // ZB
BLK_ITERS = -(-NBLK // NSUB)

ES1 = E // NSUB
NCHUNK1 = ES1 // K
ES2 = E // (NCORE * NSUB)
NCHUNK2 = ES2 // K


def _prep_body(x_ref, we_ref, be_ref, a_ref, b_ref):
    xb = x_ref[...]
    wa = we_ref[0:F, :]
    wb = we_ref[F : 2 * F, :]
    a_ref[0] = (
        jnp.dot(xb, wa - wb, preferred_element_type=jnp.float32) + be_ref[...]
    )
    b_ref[0] = jnp.dot(xb, wb, preferred_element_type=jnp.float32)


_prep_call = pl.pallas_call(
    _prep_body,
    grid=(N // R, 2),
    in_specs=[
        pl.BlockSpec((R, F), lambda i, h: (i, 0)),
        pl.BlockSpec((2 * F, F), lambda i, h: (0, h)),
        pl.BlockSpec((1, F), lambda i, h: (0, h)),
    ],
    out_specs=[
        pl.BlockSpec((1, R, F), lambda i, h: (h, i, 0)),
        pl.BlockSpec((1, R, F), lambda i, h: (h, i, 0)),
    ],
    out_shape=[
        jax.ShapeDtypeStruct((2, N, F), jnp.float32),
        jax.ShapeDtypeStruct((2, N, F), jnp.float32),
    ],
)


def _sc_eagg_body(
    src_hbm, dst_hbm, a_hbm, b_hbm, eagg_hbm,
    i_s0, i_d0, i_sa0, i_da0, i_sc0, ba0, bb0,
    i_s1, i_d1, i_sa1, i_da1, i_sc1, ba1, bb1,
    eagg_sh,
    semi0, semd0, sems0, semi1, semd1, sems1,
):
    c = lax.axis_index("c")
    s = lax.axis_index("s")
    cn = c * N

    slots = (
        (i_s0, i_d0, i_sa0, i_da0, i_sc0, ba0, bb0, semi0, semd0, sems0),
        (i_s1, i_d1, i_sa1, i_da1, i_sc1, ba1, bb1, semi1, semd1, sems1),
    )

    def idx_cp(sl, g, start):
        off = s * ES1 + g * K
        c1 = pltpu.make_async_copy(src_hbm.at[pl.ds(off, K)], sl[0], sl[7])
        c2 = pltpu.make_async_copy(dst_hbm.at[pl.ds(off, K)], sl[1], sl[7])
        if start:
            c1.start()
            c2.start()
        else:
            c1.wait()
            c2.wait()

    def adjust(sl):
        for j in range(K // 16):
            slc = pl.ds(j * 16, 16)
            vs = sl[0][slc]
            vd = sl[1][slc]
            sl[2][slc] = vs + cn
            sl[3][slc] = vd + cn
            sl[4][slc] = vd

    def gat_cp(sl, start):
        c1 = pltpu.make_async_copy(a_hbm.at[sl[3]], sl[5], sl[8])
        c2 = pltpu.make_async_copy(b_hbm.at[sl[2]], sl[6], sl[8])
        if start:
            c1.start()
            c2.start()
        else:
            c1.wait()
            c2.wait()

    def sct_start(sl):
        pltpu.async_copy(sl[6], eagg_sh.at[sl[4]], sl[9], add=True)

    def sct_wait(sl):
        pltpu.make_async_copy(sl[6], eagg_sh.at[sl[4]], sl[9]).wait()

    def relu(sl):
        def row(k, carry2):
            for j in range(F // 16):
                slc = pl.ds(j * 16, 16)
                sl[6][k, slc] = jnp.maximum(sl[5][k, slc] + sl[6][k, slc], 0.0)
            return carry2

        lax.fori_loop(0, K, row, 0)

    z16 = jnp.zeros((16,), jnp.float32)

    def zrow(i, carry):
        for j in range(F // 16):
            ba0[i, pl.ds(j * 16, 16)] = z16
        return carry

    lax.fori_loop(0, ZB, zrow, 0)

    def zblk(t, carry):
        b = s + NSUB * t

        @pl.when(b < NBLK)
        def _():
            pltpu.sync_copy(ba0, eagg_sh.at[pl.ds(b * ZB, ZB)])

        return carry

    lax.fori_loop(0, BLK_ITERS, zblk, 0)
    plsc.subcore_barrier()

    idx_cp(slots[0], 0, True)
    idx_cp(slots[1], 1, True)
    idx_cp(slots[0], 0, False)
    adjust(slots[0])
    gat_cp(slots[0], True)

    def pair(p, carry):
        for b in range(2):
            g = 2 * p + b
            sl = slots[b]
            so = slots[1 - b]

            @pl.when(g + 1 < NCHUNK1)
            def _():
                idx_cp(so, g + 1, False)

                @pl.when(g >= 1)
                def _():
                    sct_wait(so)

                adjust(so)
                gat_cp(so, True)

            gat_cp(sl, False)
            relu(sl)
            sct_start(sl)

            @pl.when(g + 2 < NCHUNK1)
            def _():
                idx_cp(sl, g + 2, True)

        return carry

    lax.fori_loop(0, NCHUNK1 // 2, pair, 0)
    sct_wait(slots[0])
    sct_wait(slots[1])
    plsc.subcore_barrier()

    def wblk(t, carry):
        b = s + NSUB * t

        @pl.when(b < NBLK)
        def _():
            r0 = b * ZB
            pltpu.sync_copy(eagg_sh.at[pl.ds(r0, ZB)], eagg_hbm.at[pl.ds(cn + r0, ZB)])

        return carry

    lax.fori_loop(0, BLK_ITERS, wblk, 0)


_sc_eagg_call = pl.kernel(
    _sc_eagg_body,
    out_type=jax.ShapeDtypeStruct((2 * N, F), jnp.float32),
    mesh=plsc.VectorSubcoreMesh(core_axis_name="c", subcore_axis_name="s"),
    scratch_types=(
        [pltpu.VMEM((K,), jnp.int32)] * 5
        + [pltpu.VMEM((K, F), jnp.float32)] * 2
        + [pltpu.VMEM((K,), jnp.int32)] * 5
        + [pltpu.VMEM((K, F), jnp.float32)] * 2
        + [pltpu.VMEM_SHARED((N, F), jnp.float32)]
        + [pltpu.SemaphoreType.DMA] * 6
    ),
)


def _sc_agg_body(
    src_hbm, dst_hbm, x_hbm, agg_hbm,
    i_s0, i_d0, i_sc0, bx0,
    i_s1, i_d1, i_sc1, bx1,
    agg_sh,
    semi0, semd0, sems0, semi1, semd1, sems1,
):
    c = lax.axis_index("c")
    s = lax.axis_index("s")
    cn = c * N

    slots = (
        (i_s0, i_d0, i_sc0, bx0, semi0, semd0, sems0),
        (i_s1, i_d1, i_sc1, bx1, semi1, semd1, sems1),
    )

    def idx_cp(sl, g, start):
        off = c * (E // NCORE) + s * ES2 + g * K
        c1 = pltpu.make_async_copy(src_hbm.at[pl.ds(off, K)], sl[0], sl[4])
        c2 = pltpu.make_async_copy(dst_hbm.at[pl.ds(off, K)], sl[1], sl[4])
        if start:
            c1.start()
            c2.start()
        else:
            c1.wait()
            c2.wait()

    def adjust(sl):
        for j in range(K // 16):
            slc = pl.ds(j * 16, 16)
            sl[2][slc] = sl[1][slc]

    def gat_cp(sl, start):
        c1 = pltpu.make_async_copy(x_hbm.at[sl[0]], sl[3], sl[5])
        if start:
            c1.start()
        else:
            c1.wait()

    def sct_start(sl):
        pltpu.async_copy(sl[3], agg_sh.at[sl[2]], sl[6], add=True)

    def sct_wait(sl):
        pltpu.make_async_copy(sl[3], agg_sh.at[sl[2]], sl[6]).wait()

    z16 = jnp.zeros((16,), jnp.float32)

    def zrow(i, carry):
        for j in range(F // 16):
            bx0[i, pl.ds(j * 16, 16)] = z16
        return carry

    lax.fori_loop(0, ZB, zrow, 0)

    def zblk(t, carry):
        b = s + NSUB * t

        @pl.when(b < NBLK)
        def _():
            pltpu.sync_copy(bx0, agg_sh.at[pl.ds(b * ZB, ZB)])

        return carry

    lax.fori_loop(0, BLK_ITERS, zblk, 0)
    plsc.subcore_barrier()

    idx_cp(slots[0], 0, True)
    idx_cp(slots[1], 1, True)
    idx_cp(slots[0], 0, False)
    adjust(slots[0])
    gat_cp(slots[0], True)

    def step(g, b):
        sl = slots[b]
        so = slots[1 - b]

        @pl.when(g + 1 < NCHUNK2)
        def _():
            idx_cp(so, g + 1, False)

            @pl.when(g >= 1)
            def _():
                sct_wait(so)

            adjust(so)
            gat_cp(so, True)

        gat_cp(sl, False)
        sct_start(sl)

        @pl.when(g + 2 < NCHUNK2)
        def _():
            idx_cp(sl, g + 2, True)

    def pair(p, carry):
        for b in range(2):
            step(2 * p + b, b)
        return carry

    lax.fori_loop(0, NCHUNK2 // 2, pair, 0)
    step(NCHUNK2 - 1, 0)
    sct_wait(slots[1])
    sct_wait(slots[0])
    plsc.subcore_barrier()

    def wblk(t, carry):
        b = s + NSUB * t

        @pl.when(b < NBLK)
        def _():
            r0 = b * ZB
            pltpu.sync_copy(agg_sh.at[pl.ds(r0, ZB)], agg_hbm.at[pl.ds(cn + r0, ZB)])

        return carry

    lax.fori_loop(0, BLK_ITERS, wblk, 0)


_sc_agg_call = pl.kernel(
    _sc_agg_body,
    out_type=jax.ShapeDtypeStruct((2 * N, F), jnp.float32),
    mesh=plsc.VectorSubcoreMesh(core_axis_name="c", subcore_axis_name="s"),
    scratch_types=(
        [pltpu.VMEM((K,), jnp.int32)] * 3
        + [pltpu.VMEM((K, F), jnp.float32)]
        + [pltpu.VMEM((K,), jnp.int32)] * 3
        + [pltpu.VMEM((K, F), jnp.float32)]
        + [pltpu.VMEM_SHARED((N, F), jnp.float32)]
        + [pltpu.SemaphoreType.DMA] * 6
    ),
)


def _head_body(
    a0_ref, a1_ref, e0_ref, e1_ref,
    w1_ref, b1_ref, w2_ref, b2_ref, w3_ref, b3_ref, w4_ref, b4_ref,
    out_ref,
):
    f32 = jnp.float32
    agg = a0_ref[...] + a1_ref[...]
    nodes = jnp.maximum(
        jnp.dot(agg, w1_ref[...], preferred_element_type=f32) + b1_ref[...],
        0.0,
    )
    edges = jnp.maximum(
        jnp.dot(e0_ref[...], w2_ref[0:F, :], preferred_element_type=f32)
        + jnp.dot(e1_ref[...], w2_ref[F : 2 * F, :], preferred_element_type=f32)
        + b2_ref[...],
        0.0,
    )
    h3 = jnp.maximum(
        jnp.dot(nodes, w3_ref[0:H, :], preferred_element_type=f32)
        + jnp.dot(edges, w3_ref[H : H + F, :], preferred_element_type=f32)
        + b3_ref[...],
        0.0,
    )
    logits = jnp.dot(h3, w4_ref[...], preferred_element_type=f32) + b4_ref[...]
    out_ref[...] = 1.0 / (1.0 + jnp.exp(-logits))


_head_call = pl.pallas_call(
    _head_body,
    grid=(N // R,),
    in_specs=[
        pl.BlockSpec((R, F), lambda i: (i, 0)),
        pl.BlockSpec((R, F), lambda i: (i + N // R, 0)),
        pl.BlockSpec((R, F), lambda i: (i, 0)),
        pl.BlockSpec((R, F), lambda i: (i + N // R, 0)),
        pl.BlockSpec((F, H), lambda i: (0, 0)),
        pl.BlockSpec((1, H), lambda i: (0, 0)),
        pl.BlockSpec((2 * F, F), lambda i: (0, 0)),
        pl.BlockSpec((1, F), lambda i: (0, 0)),
        pl.BlockSpec((H + F, 32), lambda i: (0, 0)),
        pl.BlockSpec((1, 32), lambda i: (0, 0)),
        pl.BlockSpec((32, 1), lambda i: (0, 0)),
        pl.BlockSpec((1, 1), lambda i: (0, 0)),
    ],
    out_specs=pl.BlockSpec((R, 1), lambda i: (i, 0)),
    out_shape=jax.ShapeDtypeStruct((N, 1), jnp.float32),
)


@jax.jit
def kernel(x, edge_index, e, W1, b1, We, be, W2, b2, W3, b3, W4, b4):
    del e
    src = edge_index[0]
    dst = edge_index[1]
    a2, b2t = _prep_call(x, We, be.reshape(1, H))
    eagg = _sc_eagg_call(src, dst, a2.reshape(2 * N, F), b2t.reshape(2 * N, F))
    aggp = _sc_agg_call(src, dst, x)
    out = _head_call(
        aggp, aggp, eagg, eagg,
        W1, b1.reshape(1, H),
        W2, b2.reshape(1, F),
        W3, b3.reshape(1, 32),
        W4, b4.reshape(1, 1),
    )
    return out
